# baseline (speedup 1.0000x reference)
import jax
import jax.numpy as jnp
from jax import lax
from jax.experimental import pallas as pl
from jax.experimental.pallas import tpu as pltpu

_NUM_EXPERTS = 16
_K = 2
_COEF = 0.01
_EPS = 1e-10


def _gating_body(x_ref, w_ref, gates_ref, aux_ref, imp_ref, load_ref):
    i = pl.program_id(0)
    n = pl.num_programs(0)

    @pl.when(i == 0)
    def _init():
        imp_ref[...] = jnp.zeros_like(imp_ref)
        load_ref[...] = jnp.zeros_like(load_ref)

    logits = jnp.dot(x_ref[...], w_ref[...],
                     preferred_element_type=jnp.float32)

    cols = logits.shape[1]
    tri_r = lax.broadcasted_iota(jnp.int32, (cols, cols), 0)
    tri_c = lax.broadcasted_iota(jnp.int32, (cols, cols), 1)
    tri = (tri_r <= tri_c).astype(jnp.float32)

    m1 = jnp.max(logits, axis=1, keepdims=True)
    eq1 = (logits == m1).astype(jnp.float32)
    c1 = jnp.dot(eq1, tri, preferred_element_type=jnp.float32)
    first1 = (eq1 * c1) == 1.0
    masked = jnp.where(first1, -jnp.inf, logits)
    m2 = jnp.max(masked, axis=1, keepdims=True)
    eq2 = (masked == m2).astype(jnp.float32)
    c2 = jnp.dot(eq2, tri, preferred_element_type=jnp.float32)
    first2 = (eq2 * c2) == 1.0

    d = jnp.exp(m2 - m1)
    s = 1.0 + d
    g1 = 1.0 / s
    g2 = d / s

    gates = jnp.where(first1, g1, jnp.where(first2, g2, 0.0))
    gates_ref[...] = gates

    imp_ref[...] += jnp.sum(gates, axis=0, keepdims=True)
    load_ref[...] += jnp.sum((gates > 0.0).astype(jnp.float32), axis=0,
                             keepdims=True)

    @pl.when(i == n - 1)
    def _finish():
        ne = float(_NUM_EXPERTS)
        imp = imp_ref[0, :]
        ld = load_ref[0, :]
        imp_mean = jnp.sum(imp) / ne
        ld_mean = jnp.sum(ld) / ne
        imp_var = jnp.sum((imp - imp_mean) ** 2) / (ne - 1.0)
        ld_var = jnp.sum((ld - ld_mean) ** 2) / (ne - 1.0)
        aux = _COEF * (imp_var / (imp_mean * imp_mean + _EPS)
                       + ld_var / (ld_mean * ld_mean + _EPS))
        aux_ref[0, 0] = aux


def kernel(x, w_gate):
    n_tokens, d_model = x.shape
    tile = 512
    grid = n_tokens // tile

    gates, aux = pl.pallas_call(
        _gating_body,
        grid=(grid,),
        in_specs=[
            pl.BlockSpec((tile, d_model), lambda i: (i, 0)),
            pl.BlockSpec((d_model, _NUM_EXPERTS), lambda i: (0, 0)),
        ],
        out_specs=[
            pl.BlockSpec((tile, _NUM_EXPERTS), lambda i: (i, 0)),
            pl.BlockSpec(memory_space=pltpu.SMEM),
        ],
        out_shape=[
            jax.ShapeDtypeStruct((n_tokens, _NUM_EXPERTS), jnp.float32),
            jax.ShapeDtypeStruct((1, 1), jnp.float32),
        ],
        scratch_shapes=[
            pltpu.VMEM((1, _NUM_EXPERTS), jnp.float32),
            pltpu.VMEM((1, _NUM_EXPERTS), jnp.float32),
        ],
    )(x, w_gate)
    return gates, aux.reshape(())


# speedup vs baseline: 1.3253x; 1.3253x over previous
import jax
import jax.numpy as jnp
from jax import lax
from jax.experimental import pallas as pl
from jax.experimental.pallas import tpu as pltpu

_NUM_EXPERTS = 16
_K = 2
_COEF = 0.01
_EPS = 1e-10


def _gating_body(x_ref, w_ref, gates_ref, aux_ref, imp_ref, load_ref):
    i = pl.program_id(0)
    n = pl.num_programs(0)

    @pl.when(i == 0)
    def _init():
        imp_ref[...] = jnp.zeros_like(imp_ref)
        load_ref[...] = jnp.zeros_like(load_ref)

    logits = jnp.dot(x_ref[...], w_ref[...],
                     preferred_element_type=jnp.float32)

    cols = logits.shape[1]
    tri_r = lax.broadcasted_iota(jnp.int32, (cols, cols), 0)
    tri_c = lax.broadcasted_iota(jnp.int32, (cols, cols), 1)
    tri = (tri_r <= tri_c).astype(jnp.float32)

    m1 = jnp.max(logits, axis=1, keepdims=True)
    eq1 = (logits == m1).astype(jnp.float32)
    c1 = jnp.dot(eq1, tri, preferred_element_type=jnp.float32)
    first1 = (eq1 * c1) == 1.0
    masked = jnp.where(first1, -jnp.inf, logits)
    m2 = jnp.max(masked, axis=1, keepdims=True)
    eq2 = (masked == m2).astype(jnp.float32)
    c2 = jnp.dot(eq2, tri, preferred_element_type=jnp.float32)
    first2 = (eq2 * c2) == 1.0

    d = jnp.exp(m2 - m1)
    s = 1.0 + d
    g1 = 1.0 / s
    g2 = d / s

    gates = jnp.where(first1, g1, jnp.where(first2, g2, 0.0))
    gates_ref[...] = gates

    imp_ref[...] += jnp.sum(gates, axis=0, keepdims=True)
    load_ref[...] += jnp.sum((gates > 0.0).astype(jnp.float32), axis=0,
                             keepdims=True)

    @pl.when(i == n - 1)
    def _finish():
        ne = float(_NUM_EXPERTS)
        imp = imp_ref[0, :]
        ld = load_ref[0, :]
        imp_mean = jnp.sum(imp) / ne
        ld_mean = jnp.sum(ld) / ne
        imp_var = jnp.sum((imp - imp_mean) ** 2) / (ne - 1.0)
        ld_var = jnp.sum((ld - ld_mean) ** 2) / (ne - 1.0)
        aux = _COEF * (imp_var / (imp_mean * imp_mean + _EPS)
                       + ld_var / (ld_mean * ld_mean + _EPS))
        aux_ref[0, 0] = aux


def kernel(x, w_gate):
    n_tokens, d_model = x.shape
    tile = 2048
    grid = n_tokens // tile

    gates, aux = pl.pallas_call(
        _gating_body,
        grid=(grid,),
        in_specs=[
            pl.BlockSpec((tile, d_model), lambda i: (i, 0)),
            pl.BlockSpec((d_model, _NUM_EXPERTS), lambda i: (0, 0)),
        ],
        out_specs=[
            pl.BlockSpec((tile, _NUM_EXPERTS), lambda i: (i, 0)),
            pl.BlockSpec(memory_space=pltpu.SMEM),
        ],
        out_shape=[
            jax.ShapeDtypeStruct((n_tokens, _NUM_EXPERTS), jnp.float32),
            jax.ShapeDtypeStruct((1, 1), jnp.float32),
        ],
        scratch_shapes=[
            pltpu.VMEM((1, _NUM_EXPERTS), jnp.float32),
            pltpu.VMEM((1, _NUM_EXPERTS), jnp.float32),
        ],
    )(x, w_gate)
    return gates, aux.reshape(())
